# baseline (speedup 1.0000x reference)
import functools

import jax
import jax.numpy as jnp
from jax import lax
from jax.experimental import pallas as pl
from jax.experimental.pallas import tpu as pltpu

B, S, DIM = 1, 2048, 1024
NUM_EXPERTS = 8
HIDDEN = 2048
TOP_K = 2
T = B * S
TM = 128
L = T * TOP_K + NUM_EXPERTS * TM
NS = L // TM


def _moe_kernel(step_group, step_valid, scores_ref, x_ref, ids_ref, wts_ref,
                w1_ref, w3_ref, w2_ref, out_ref, aux_ref):
    s = pl.program_id(0)

    @pl.when(s == 0)
    def _prologue():
        out_ref[...] = jnp.zeros_like(out_ref)
        sc = scores_ref[...]
        m = jnp.max(sc, axis=1, keepdims=True)
        p = jnp.exp(sc - m)
        probs = p / jnp.sum(p, axis=1, keepdims=True)
        usage = jnp.mean(probs, axis=0, keepdims=True)
        aux_ref[...] = NUM_EXPERTS * jnp.sum(usage * usage, axis=1,
                                             keepdims=True)

    @pl.when(step_valid[s] != 0)
    def _body():
        ids_col = ids_ref[0]
        w_col = wts_ref[0]
        iota = lax.broadcasted_iota(jnp.int32, (TM, T), 1)
        P = (iota == ids_col).astype(jnp.bfloat16)
        xs = lax.dot_general(P, x_ref[...], (((1,), (0,)), ((), ())),
                             preferred_element_type=jnp.float32)
        xs = xs.astype(jnp.bfloat16)
        w1 = w1_ref[0]
        w3 = w3_ref[0]
        w2 = w2_ref[0]
        h1 = lax.dot_general(xs, w1, (((1,), (1,)), ((), ())),
                             preferred_element_type=jnp.float32)
        h3 = lax.dot_general(xs, w3, (((1,), (1,)), ((), ())),
                             preferred_element_type=jnp.float32)
        h = (h1 * jax.nn.sigmoid(h1) * h3).astype(jnp.bfloat16)
        y = lax.dot_general(h, w2, (((1,), (1,)), ((), ())),
                            preferred_element_type=jnp.float32)
        y = (y * w_col).astype(jnp.bfloat16)
        out_ref[...] += lax.dot_general(P, y, (((0,), (0,)), ((), ())),
                                        preferred_element_type=jnp.float32)


@functools.partial(jax.jit, static_argnums=())
def kernel(x, Wg, W1, W2, W3):
    b, s_len, d = x.shape
    gate_scores = jnp.einsum('bsd,ed->bse', x, Wg)
    top_k_values, top_k_indices = jax.lax.top_k(gate_scores, TOP_K)
    top_k_weights = jax.nn.softmax(top_k_values, axis=-1)

    idx_flat = top_k_indices.reshape(-1).astype(jnp.int32)
    w_flat = top_k_weights.reshape(-1)

    order = jnp.argsort(idx_flat, stable=True)
    sorted_e = idx_flat[order]
    sizes = jnp.bincount(idx_flat, length=NUM_EXPERTS)
    start = jnp.concatenate([jnp.zeros((1,), sizes.dtype),
                             jnp.cumsum(sizes)[:-1]])
    padded = ((sizes + TM - 1) // TM) * TM
    pstart = jnp.concatenate([jnp.zeros((1,), padded.dtype),
                              jnp.cumsum(padded)[:-1]])
    ranks = jnp.arange(T * TOP_K) - start[sorted_e]
    dest = pstart[sorted_e] + ranks
    tok_ids = jnp.zeros((L,), jnp.int32).at[dest].set(
        (order // TOP_K).astype(jnp.int32))
    wts = jnp.zeros((L,), jnp.float32).at[dest].set(w_flat[order])

    ptiles_end = (jnp.cumsum(padded) // TM).astype(jnp.int32)
    num_real = ptiles_end[-1]
    s_arr = jnp.arange(NS, dtype=jnp.int32)
    step_group = jnp.minimum(
        jnp.searchsorted(ptiles_end, s_arr, side='right').astype(jnp.int32),
        NUM_EXPERTS - 1)
    step_valid = (s_arr < num_real).astype(jnp.int32)

    x_flat = x.reshape(T, d)
    xb = x_flat.astype(jnp.bfloat16)
    scores2d = gate_scores.reshape(T, NUM_EXPERTS)
    ids3 = tok_ids.reshape(NS, TM, 1)
    wts3 = wts.reshape(NS, TM, 1)

    grid_spec = pltpu.PrefetchScalarGridSpec(
        num_scalar_prefetch=2,
        grid=(NS,),
        in_specs=[
            pl.BlockSpec((T, NUM_EXPERTS), lambda i, sg, sv: (0, 0)),
            pl.BlockSpec((T, d), lambda i, sg, sv: (0, 0)),
            pl.BlockSpec((1, TM, 1), lambda i, sg, sv: (i, 0, 0)),
            pl.BlockSpec((1, TM, 1), lambda i, sg, sv: (i, 0, 0)),
            pl.BlockSpec((1, HIDDEN, d), lambda i, sg, sv: (sg[i], 0, 0)),
            pl.BlockSpec((1, HIDDEN, d), lambda i, sg, sv: (sg[i], 0, 0)),
            pl.BlockSpec((1, d, HIDDEN), lambda i, sg, sv: (sg[i], 0, 0)),
        ],
        out_specs=[
            pl.BlockSpec((T, d), lambda i, sg, sv: (0, 0)),
            pl.BlockSpec((1, 1), lambda i, sg, sv: (0, 0)),
        ],
    )
    out, aux = pl.pallas_call(
        _moe_kernel,
        grid_spec=grid_spec,
        out_shape=[
            jax.ShapeDtypeStruct((T, d), jnp.float32),
            jax.ShapeDtypeStruct((1, 1), jnp.float32),
        ],
        compiler_params=pltpu.CompilerParams(
            dimension_semantics=("arbitrary",)),
    )(step_group, step_valid, scores2d, xb, ids3, wts3,
      W1.astype(jnp.bfloat16), W3.astype(jnp.bfloat16),
      W2.astype(jnp.bfloat16))
    return out.reshape(b, s_len, d), aux[0, 0]
